# baseline (speedup 1.0000x reference)
import dataclasses
import functools

import jax
import jax.numpy as jnp
from jax import lax
from jax.experimental import pallas as pl
from jax.experimental.pallas import tpu as pltpu
from jax.experimental.pallas import tpu_sc as plsc

_B, _C, _H, _W = 4, 64, 384, 384
_HB = 128
_NH = _H // _HB
_SPLITS = (16, 32, 48)
_K = 2 * len(_SPLITS)
_RB = 8
_Q = _C // 4


_CB = 16


def _sums_body(x_ref, s_ref):
    s_ref[0, 0, 0, :] = jnp.sum(x_ref[0], axis=(1, 2))


_L = 16


def _sc_perm_body(s_hbm, p_hbm, sv, pv):
    wid = lax.axis_index("s") * 2 + lax.axis_index("c")

    @pl.when(wid < _B)
    def _():
        pltpu.sync_copy(s_hbm.at[wid], sv)
        for q in range(_C // _L):
            my = sv[pl.ds(_L * q, _L)]
            gc = _L * q + lax.iota(jnp.int32, _L)
            rank = jnp.zeros((_L,), jnp.int32)
            for jq in range(_C // _L):
                other = sv[pl.ds(_L * jq, _L)]
                for k in range(_L):
                    bk = other.at[jnp.full((_L,), k, jnp.int32)].get(
                        mode="promise_in_bounds")
                    gj = _L * jq + k
                    ahead = (bk < my) | ((bk == my) & (gj < gc))
                    rank = rank + ahead.astype(jnp.int32)
            plsc.store_scatter(pv, [rank], gc)
        pltpu.sync_copy(pv, p_hbm.at[wid])


_C16_S, _C16_G = 1.0 / 48, -(1.0 / 16 + 1.0 / 48)
_C32_S, _C32_G = 1.0 / 32, -(1.0 / 16)
_C48_S, _C48_G = -(1.0 / 48), (1.0 / 48 + 1.0 / 16)


def _rcp_norm_body(p_ref, x_ref, o_ref, y_scr, mm_scr):
    ph = pl.program_id(0)
    b = pl.program_id(1)
    h = pl.program_id(2)

    @pl.when(ph == 0)
    def _():
        vmins, vmaxs = [], []
        for hs in range(0, _HB, _RB):
            q = [jnp.zeros((_RB, _W), jnp.float32) for _ in range(4)]
            g = [jnp.zeros((_RB, _W), jnp.float32) for _ in range(4)]
            for c in range(_C):
                q[c // _Q] = q[c // _Q] + x_ref[0, c, hs:hs + _RB, :]
            for r in range(_C):
                pc = p_ref[b, 0, r]
                g[r // _Q] = g[r // _Q] + x_ref[0, pc, hs:hs + _RB, :]
            s = (q[0] + q[1]) + (q[2] + q[3])
            g01 = g[0] + g[1]
            q01 = q[0] + q[1]
            maps = [
                _C16_S * s + _C16_G * g[0],
                _C32_S * s + _C32_G * g01,
                _C48_S * s + _C48_G * g[3],
                _C16_S * s + _C16_G * q[0],
                _C32_S * s + _C32_G * q01,
                _C48_S * s + _C48_G * q[3],
            ]
            for k in range(_K):
                y_scr[b, k, pl.ds(h * _HB + hs, _RB), :] = maps[k]
            vmins.append(jnp.stack([jnp.min(a) for a in maps]))
            vmaxs.append(jnp.stack([jnp.max(a) for a in maps]))
        vmin = functools.reduce(jnp.minimum, vmins)
        vmax = functools.reduce(jnp.maximum, vmaxs)

        @pl.when(h == 0)
        def _():
            mm_scr[b, 0, :] = vmin
            mm_scr[b, 1, :] = vmax

        @pl.when(h != 0)
        def _():
            mm_scr[b, 0, :] = jnp.minimum(mm_scr[b, 0, :], vmin)
            mm_scr[b, 1, :] = jnp.maximum(mm_scr[b, 1, :], vmax)

    @pl.when(ph == 1)
    def _():
        mn = mm_scr[b, 0, :]
        mx = mm_scr[b, 1, :]
        yb = y_scr[b, :, pl.ds(h * _HB, _HB), :]
        o_ref[0] = (yb - mn[:, None, None]) / (mx - mn + 1e-8)[:, None, None]


@jax.jit
def kernel(x):
    f32 = jnp.float32

    sums = pl.pallas_call(
        _sums_body,
        grid=(_B, _C // _CB),
        in_specs=[pl.BlockSpec((1, _CB, _H, _W), lambda b, cq: (b, cq, 0, 0))],
        out_specs=pl.BlockSpec((1, 1, 1, _CB), lambda b, cq: (b, cq, 0, 0)),
        out_shape=jax.ShapeDtypeStruct((_B, _C // _CB, 1, _CB), f32),
    )(x)

    cp = pltpu.CompilerParams()
    if "needs_layout_passes" in pltpu.CompilerParams.__dataclass_fields__:
        cp = dataclasses.replace(cp, needs_layout_passes=False)
    perm2d = pl.kernel(
        _sc_perm_body,
        out_type=jax.ShapeDtypeStruct((_B, _C), jnp.int32),
        mesh=plsc.VectorSubcoreMesh(core_axis_name="c", subcore_axis_name="s"),
        scratch_types=[
            pltpu.VMEM((_C,), f32),
            pltpu.VMEM((_C,), jnp.int32),
        ],
        compiler_params=cp,
    )(sums.reshape(_B, _C))
    perm = perm2d.reshape(_B, 1, _C)

    out = pl.pallas_call(
        _rcp_norm_body,
        grid=(2, _B, _NH),
        in_specs=[
            pl.BlockSpec(memory_space=pltpu.SMEM),
            pl.BlockSpec(
                (1, _C, _HB, _W),
                lambda ph, b, h: (
                    jnp.where(ph == 0, b, _B - 1),
                    0,
                    jnp.where(ph == 0, h, _NH - 1),
                    0,
                ),
            ),
        ],
        out_specs=pl.BlockSpec(
            (1, _K, _HB, _W),
            lambda ph, b, h: (
                jnp.where(ph == 0, 0, b),
                0,
                jnp.where(ph == 0, 0, h),
                0,
            ),
        ),
        out_shape=jax.ShapeDtypeStruct((_B, _K, _H, _W), f32),
        scratch_shapes=[
            pltpu.VMEM((_B, _K, _H, _W), f32),
            pltpu.VMEM((_B, 2, _K), f32),
        ],
    )(perm, x)

    return out


# speedup vs baseline: 1.0217x; 1.0217x over previous
import dataclasses
import functools

import jax
import jax.numpy as jnp
from jax import lax
from jax.experimental import pallas as pl
from jax.experimental.pallas import tpu as pltpu
from jax.experimental.pallas import tpu_sc as plsc

_B, _C, _H, _W = 4, 64, 384, 384
_HB = 128
_NH = _H // _HB
_SPLITS = (16, 32, 48)
_K = 2 * len(_SPLITS)
_RB = 8
_Q = _C // 4


def _sums_body(x_ref, s_ref):
    h = pl.program_id(1)
    part = jnp.sum(x_ref[0], axis=(1, 2))

    @pl.when(h == 0)
    def _():
        s_ref[0, 0, :] = part

    @pl.when(h != 0)
    def _():
        s_ref[0, 0, :] = s_ref[0, 0, :] + part


_L = 16


def _sc_perm_body(s_hbm, p_hbm, sv, pv):
    wid = lax.axis_index("s") * 2 + lax.axis_index("c")

    @pl.when(wid < _B)
    def _():
        pltpu.sync_copy(s_hbm.at[wid], sv)
        for q in range(_C // _L):
            my = sv[pl.ds(_L * q, _L)]
            gc = _L * q + lax.iota(jnp.int32, _L)
            rank = jnp.zeros((_L,), jnp.int32)
            for jq in range(_C // _L):
                other = sv[pl.ds(_L * jq, _L)]
                for k in range(_L):
                    bk = other.at[jnp.full((_L,), k, jnp.int32)].get(
                        mode="promise_in_bounds")
                    gj = _L * jq + k
                    ahead = (bk < my) | ((bk == my) & (gj < gc))
                    rank = rank + ahead.astype(jnp.int32)
            plsc.store_scatter(pv, [rank], gc)
        pltpu.sync_copy(pv, p_hbm.at[wid])


_C16_S, _C16_G = 1.0 / 48, -(1.0 / 16 + 1.0 / 48)
_C32_S, _C32_G = 1.0 / 32, -(1.0 / 16)
_C48_S, _C48_G = -(1.0 / 48), (1.0 / 48 + 1.0 / 16)


def _rcp_norm_body(p_ref, x_ref, o_ref, y_scr, mm_scr):
    ph = pl.program_id(0)
    b = pl.program_id(1)
    h = pl.program_id(2)

    @pl.when(ph == 0)
    def _():
        vmins, vmaxs = [], []
        for hs in range(0, _HB, _RB):
            q = [jnp.zeros((_RB, _W), jnp.float32) for _ in range(4)]
            g = [jnp.zeros((_RB, _W), jnp.float32) for _ in range(4)]
            for c in range(_C):
                q[c // _Q] = q[c // _Q] + x_ref[0, c, hs:hs + _RB, :]
            for r in range(_C):
                pc = p_ref[b, 0, r]
                g[r // _Q] = g[r // _Q] + x_ref[0, pc, hs:hs + _RB, :]
            s = (q[0] + q[1]) + (q[2] + q[3])
            g01 = g[0] + g[1]
            q01 = q[0] + q[1]
            maps = [
                _C16_S * s + _C16_G * g[0],
                _C32_S * s + _C32_G * g01,
                _C48_S * s + _C48_G * g[3],
                _C16_S * s + _C16_G * q[0],
                _C32_S * s + _C32_G * q01,
                _C48_S * s + _C48_G * q[3],
            ]
            for k in range(_K):
                y_scr[b, k, pl.ds(h * _HB + hs, _RB), :] = maps[k]
            vmins.append(jnp.stack([jnp.min(a) for a in maps]))
            vmaxs.append(jnp.stack([jnp.max(a) for a in maps]))
        vmin = functools.reduce(jnp.minimum, vmins)
        vmax = functools.reduce(jnp.maximum, vmaxs)

        @pl.when(h == 0)
        def _():
            mm_scr[b, 0, :] = vmin
            mm_scr[b, 1, :] = vmax

        @pl.when(h != 0)
        def _():
            mm_scr[b, 0, :] = jnp.minimum(mm_scr[b, 0, :], vmin)
            mm_scr[b, 1, :] = jnp.maximum(mm_scr[b, 1, :], vmax)

    @pl.when(ph == 1)
    def _():
        mn = mm_scr[b, 0, :]
        mx = mm_scr[b, 1, :]
        yb = y_scr[b, :, pl.ds(h * _HB, _HB), :]
        o_ref[0] = (yb - mn[:, None, None]) / (mx - mn + 1e-8)[:, None, None]


@jax.jit
def kernel(x):
    f32 = jnp.float32

    sums = pl.pallas_call(
        _sums_body,
        grid=(_B, _NH),
        in_specs=[pl.BlockSpec((1, _C, _HB, _W), lambda b, h: (b, 0, h, 0))],
        out_specs=pl.BlockSpec((1, 1, _C), lambda b, h: (b, 0, 0)),
        out_shape=jax.ShapeDtypeStruct((_B, 1, _C), f32),
    )(x)

    cp = pltpu.CompilerParams()
    if "needs_layout_passes" in pltpu.CompilerParams.__dataclass_fields__:
        cp = dataclasses.replace(cp, needs_layout_passes=False)
    perm2d = pl.kernel(
        _sc_perm_body,
        out_type=jax.ShapeDtypeStruct((_B, _C), jnp.int32),
        mesh=plsc.VectorSubcoreMesh(core_axis_name="c", subcore_axis_name="s"),
        scratch_types=[
            pltpu.VMEM((_C,), f32),
            pltpu.VMEM((_C,), jnp.int32),
        ],
        compiler_params=cp,
    )(sums.reshape(_B, _C))
    perm = perm2d.reshape(_B, 1, _C)

    out = pl.pallas_call(
        _rcp_norm_body,
        grid=(2, _B, _NH),
        in_specs=[
            pl.BlockSpec(memory_space=pltpu.SMEM),
            pl.BlockSpec(
                (1, _C, _HB, _W),
                lambda ph, b, h: (
                    jnp.where(ph == 0, b, _B - 1),
                    0,
                    jnp.where(ph == 0, h, _NH - 1),
                    0,
                ),
            ),
        ],
        out_specs=pl.BlockSpec(
            (1, _K, _HB, _W),
            lambda ph, b, h: (
                jnp.where(ph == 0, 0, b),
                0,
                jnp.where(ph == 0, 0, h),
                0,
            ),
        ),
        out_shape=jax.ShapeDtypeStruct((_B, _K, _H, _W), f32),
        scratch_shapes=[
            pltpu.VMEM((_B, _K, _H, _W), f32),
            pltpu.VMEM((_B, 2, _K), f32),
        ],
    )(perm, x)

    return out
